# baseline (speedup 1.0000x reference)
import functools

import jax
import jax.numpy as jnp
from jax import lax
from jax.experimental import pallas as pl
from jax.experimental.pallas import tpu as pltpu
from jax.experimental.pallas import tpu_sc as plsc

_INFO = plsc.get_sparse_core_info()
_NC = _INFO.num_cores
_NS = _INFO.num_subcores
_NW = _NC * _NS


@functools.partial(
    jax.jit, static_argnames=("n_per_w", "chunk", "depth", "b", "h")
)
def _sc_gather(idx, weight, *, n_per_w, chunk, depth, b, h):
    n_total, = idx.shape
    _, d = weight.shape
    n_chunks = n_per_w // chunk
    planes = chunk // h
    mesh = plsc.VectorSubcoreMesh(core_axis_name="c", subcore_axis_name="s")

    @functools.partial(
        pl.kernel,
        mesh=mesh,
        out_type=jax.ShapeDtypeStruct((b, h, d), jnp.float32),
        scratch_types=[
            pltpu.VMEM((n_per_w,), jnp.int32),
            *[pltpu.VMEM((chunk, d), jnp.float32) for _ in range(depth)],
            *[pltpu.SemaphoreType.DMA for _ in range(depth)],
        ],
        compiler_params=pltpu.CompilerParams(use_tc_tiling_on_sc=False),
    )
    def k(idx_hbm, table_hbm, out_hbm, idx_v, *bufs):
        rows_b = bufs[:depth]
        sems = bufs[depth:]
        wid = lax.axis_index("s") * _NC + lax.axis_index("c")
        w_base = wid * n_per_w

        pltpu.sync_copy(idx_hbm.at[pl.ds(w_base, n_per_w)], idx_v)

        def start(c, s):
            pltpu.async_copy(
                table_hbm.at[idx_v.at[pl.ds(c * chunk, chunk)]],
                rows_b[s], sems[s],
            )

        def wait(s):
            pltpu.make_async_copy(
                table_hbm.at[idx_v.at[pl.ds(0, chunk)]], rows_b[s], sems[s]
            ).wait()

        for s in range(min(depth, n_chunks)):
            start(s, s)
        for c in range(n_chunks):
            s = c % depth
            wait(s)
            b0 = (wid * n_chunks + c) * planes
            for p in range(planes):
                pltpu.sync_copy(
                    rows_b[s].at[pl.ds(p * h, h)], out_hbm.at[b0 + p]
                )
            if c + depth < n_chunks:
                start(c + depth, s)

    return k(idx, weight)


def kernel(x, weight):
    b, h = x.shape
    v, d = weight.shape
    n = b * h
    idx = x.reshape(n).astype(jnp.int32)
    n_per_w = n // _NW
    return _sc_gather(
        idx, weight, n_per_w=n_per_w, chunk=400, depth=4, b=b, h=h
    )
